# Initial kernel scaffold; baseline (speedup 1.0000x reference)
import jax
import jax.numpy as jnp
from jax.experimental import pallas as pl


def kernel(node_features, edge_index, initial_edge_features, W_ef, b_ef, bias0, bias1):
    raise NotImplementedError("write your pallas kernel here")


# speedup vs baseline: 4.6105x; 4.6105x over previous
import functools

import jax
import jax.numpy as jnp
from jax import lax
from jax.experimental import pallas as pl
from jax.experimental.pallas import tpu as pltpu
from jax.experimental.pallas import tpu_sc as plsc

HID = 32
D_EDGE = 16
N_NODES = 10000
CHUNK = 128
EP = 163840
G_TILES = 32
S_TILES = 16
G_CH = EP // G_TILES // CHUNK
S_CH = EP // S_TILES // CHUNK
NP = 10016
NPT = NP // S_TILES
NT = N_NODES // S_TILES


NBUF = 8


def _gather_kernel(h_hbm, src_hbm, out_hbm, idx_v, bufs, gsems, wsems):
    wid = lax.axis_index("s") * 2 + lax.axis_index("c")
    base = wid * G_CH
    pltpu.sync_copy(src_hbm.at[pl.ds(base, G_CH)], idx_v)

    for b in range(NBUF):
        pltpu.async_copy(h_hbm.at[idx_v.at[b]], bufs.at[b], gsems.at[b])

    def body(g, _):
        for b in range(NBUF):
            j = g * NBUF + b
            dst = out_hbm.at[pl.ds((base + j) * CHUNK, CHUNK)]
            pltpu.make_async_copy(h_hbm.at[idx_v.at[j]], bufs.at[b],
                                  gsems.at[b]).wait()
            pltpu.async_copy(bufs.at[b], dst, wsems.at[b])

            @pl.when(j + NBUF < G_CH)
            def _():
                pltpu.make_async_copy(bufs.at[b], dst, wsems.at[b]).wait()
                pltpu.async_copy(h_hbm.at[idx_v.at[j + NBUF]], bufs.at[b],
                                 gsems.at[b])
        return 0

    lax.fori_loop(0, G_CH // NBUF, body, 0)
    for b in range(NBUF):
        pltpu.make_async_copy(
            bufs.at[b], out_hbm.at[pl.ds(base * CHUNK, CHUNK)],
            wsems.at[b]).wait()


_sc_gather = functools.partial(
    pl.kernel,
    mesh=plsc.VectorSubcoreMesh(core_axis_name="c", subcore_axis_name="s"),
    compiler_params=pltpu.CompilerParams(use_tc_tiling_on_sc=False),
    out_type=jax.ShapeDtypeStruct((EP, HID), jnp.float32),
    scratch_types=[
        pltpu.VMEM((G_CH, CHUNK), jnp.int32),
        pltpu.VMEM((NBUF, CHUNK, HID), jnp.float32),
        pltpu.SemaphoreType.DMA((NBUF,)),
        pltpu.SemaphoreType.DMA((NBUF,)),
    ],
)(_gather_kernel)


def _scatter_kernel(m_hbm, dst_hbm, h_hbm, bias_hbm, out_hbm,
                    idx_v, buf, acc_v, h_v, bias_v, agg_sh, msems):
    sid = lax.axis_index("s")
    pltpu.sync_copy(dst_hbm.at[pl.ds(sid * S_CH, S_CH)], idx_v)

    z16 = jnp.zeros((16,), jnp.float32)

    def zbody(i, _):
        acc_v[i, pl.ds(0, 16)] = z16
        acc_v[i, pl.ds(16, 16)] = z16
        return 0

    lax.fori_loop(0, NPT, zbody, 0)
    pltpu.sync_copy(acc_v, agg_sh.at[pl.ds(sid * NPT, NPT)])
    plsc.subcore_barrier()

    for b in range(NBUF):
        pltpu.async_copy(m_hbm.at[pl.ds((sid * S_CH + b) * CHUNK, CHUNK)],
                         buf.at[b], msems.at[b])

    def sbody(g, _):
        for b in range(NBUF):
            j = g * NBUF + b
            src = m_hbm.at[pl.ds((sid * S_CH + j) * CHUNK, CHUNK)]
            pltpu.make_async_copy(src, buf.at[b], msems.at[b]).wait()
            pltpu.sync_copy(buf.at[b], agg_sh.at[idx_v.at[j]], add=True)

            @pl.when(j + NBUF < S_CH)
            def _():
                pltpu.async_copy(
                    m_hbm.at[pl.ds((sid * S_CH + j + NBUF) * CHUNK, CHUNK)],
                    buf.at[b], msems.at[b])
        return 0

    lax.fori_loop(0, S_CH // NBUF, sbody, 0)
    plsc.subcore_barrier()

    pltpu.sync_copy(agg_sh.at[pl.ds(sid * NT, NT)], acc_v.at[pl.ds(0, NT)])
    pltpu.sync_copy(h_hbm.at[pl.ds(sid * NT, NT)], h_v)
    pltpu.sync_copy(bias_hbm, bias_v)
    b0 = bias_v[pl.ds(0, 16)]
    b1 = bias_v[pl.ds(16, 16)]

    def ubody(i, _):
        acc_v[i, pl.ds(0, 16)] = acc_v[i, pl.ds(0, 16)] + h_v[i, pl.ds(0, 16)] + b0
        acc_v[i, pl.ds(16, 16)] = acc_v[i, pl.ds(16, 16)] + h_v[i, pl.ds(16, 16)] + b1
        return 0

    lax.fori_loop(0, NT, ubody, 0)
    pltpu.sync_copy(acc_v.at[pl.ds(0, NT)], out_hbm.at[pl.ds(sid * NT, NT)])


_sc_scatter = functools.partial(
    pl.kernel,
    mesh=plsc.VectorSubcoreMesh(core_axis_name="c", subcore_axis_name="s",
                                num_cores=1),
    compiler_params=pltpu.CompilerParams(use_tc_tiling_on_sc=False),
    out_type=jax.ShapeDtypeStruct((N_NODES, HID), jnp.float32),
    scratch_types=[
        pltpu.VMEM((S_CH, CHUNK), jnp.int32),
        pltpu.VMEM((NBUF, CHUNK, HID), jnp.float32),
        pltpu.VMEM((NPT, HID), jnp.float32),
        pltpu.VMEM((NT, HID), jnp.float32),
        pltpu.VMEM((HID,), jnp.float32),
        pltpu.VMEM_SHARED((NP, HID), jnp.float32),
        pltpu.SemaphoreType.DMA((NBUF,)),
    ],
)(_scatter_kernel)


def _mm_body(hs_ref, ef_ref, wcat_ref, r_ref, s_ref, bmat_ref, out_ref):
    hs8 = hs_ref[...]
    ef8 = ef_ref[...]
    u = jnp.dot(hs8, wcat_ref[...], preferred_element_type=jnp.float32)
    efrep = jnp.dot(ef8, r_ref[...], preferred_element_type=jnp.float32)
    out_ref[...] = (
        jnp.dot(u * efrep, s_ref[...], preferred_element_type=jnp.float32)
        + jnp.dot(hs8, bmat_ref[...], preferred_element_type=jnp.float32))


TE = 8192
TB = TE // 4


def _tc_matmul(hs4, ef4, wcat4, r4, s4, bmat4):
    return pl.pallas_call(
        _mm_body,
        grid=(EP // TE,),
        in_specs=[
            pl.BlockSpec((TB, 128), lambda i: (i, 0)),
            pl.BlockSpec((TB, 64), lambda i: (i, 0)),
            pl.BlockSpec((128, 2048), lambda i: (0, 0)),
            pl.BlockSpec((64, 2048), lambda i: (0, 0)),
            pl.BlockSpec((2048, 128), lambda i: (0, 0)),
            pl.BlockSpec((128, 128), lambda i: (0, 0)),
        ],
        out_specs=pl.BlockSpec((TB, 128), lambda i: (i, 0)),
        out_shape=jax.ShapeDtypeStruct((EP // 4, 128), jnp.float32),
    )(hs4, ef4, wcat4, r4, s4, bmat4)


def kernel(node_features, edge_index, initial_edge_features, W_ef, b_ef,
           bias0, bias1):
    E = initial_edge_features.shape[0]
    src = edge_index[0].astype(jnp.int32)
    dst = edge_index[1].astype(jnp.int32)
    src2d = jnp.pad(src, (0, EP - E)).reshape(EP // CHUNK, CHUNK)
    dst2d = jnp.pad(dst, (0, EP - E),
                    constant_values=N_NODES).reshape(EP // CHUNK, CHUNK)
    ef4 = jnp.pad(initial_edge_features.reshape(E // 4, 64),
                  ((0, (EP - E) // 4), (0, 0)))
    wcat = W_ef.reshape(D_EDGE, HID, HID).transpose(1, 0, 2).reshape(HID, 512)
    eye_d = jnp.eye(D_EDGE, dtype=jnp.float32)
    eye_h = jnp.eye(HID, dtype=jnp.float32)
    eye4 = jnp.eye(4, dtype=jnp.float32)
    rmat = jnp.repeat(eye_d, HID, axis=1).reshape(D_EDGE, 512)
    smat = jnp.tile(eye_h, (D_EDGE, 1))
    bmat = b_ef.reshape(HID, HID)
    wcat4 = jnp.kron(eye4, wcat)
    r4 = jnp.kron(eye4, rmat)
    s4 = jnp.kron(eye4, smat)
    bmat4 = jnp.kron(eye4, bmat)

    h = node_features
    for bias in (bias0, bias1):
        hs = _sc_gather(h, src2d)
        m4 = _tc_matmul(hs.reshape(EP // 4, 128), ef4, wcat4, r4, s4, bmat4)
        h = _sc_scatter(m4.reshape(EP, HID), dst2d, h, bias)
    return h


# speedup vs baseline: 5.2773x; 1.1446x over previous
import functools

import jax
import jax.numpy as jnp
from jax import lax
from jax.experimental import pallas as pl
from jax.experimental.pallas import tpu as pltpu
from jax.experimental.pallas import tpu_sc as plsc

HID = 32
D_EDGE = 16
N_NODES = 10000
CHUNK = 128
EP = 163840
G_TILES = 32
S_TILES = 16
G_CH = EP // G_TILES // CHUNK
S_CH = EP // S_TILES // CHUNK
NP = 10016
NPT = NP // S_TILES
NT = N_NODES // S_TILES


NBUF = 8


def _gather_kernel(h_hbm, src_hbm, out_hbm, idx_v, bufs, h_sh, wsems):
    sid = lax.axis_index("s")
    wid = sid * 2 + lax.axis_index("c")
    base = wid * G_CH
    pltpu.sync_copy(src_hbm.at[pl.ds(base, G_CH)], idx_v)
    pltpu.sync_copy(h_hbm.at[pl.ds(sid * NT, NT)], h_sh.at[pl.ds(sid * NT, NT)])
    plsc.subcore_barrier()

    def body(g, _):
        for b in range(NBUF):
            j = g * NBUF + b
            dst = out_hbm.at[pl.ds((base + j) * CHUNK, CHUNK)]

            @pl.when(j >= NBUF)
            def _():
                pltpu.make_async_copy(bufs.at[b], dst, wsems.at[b]).wait()

            pltpu.sync_copy(h_sh.at[idx_v.at[j]], bufs.at[b])
            pltpu.async_copy(bufs.at[b], dst, wsems.at[b])
        return 0

    lax.fori_loop(0, G_CH // NBUF, body, 0)
    for b in range(NBUF):
        pltpu.make_async_copy(
            bufs.at[b], out_hbm.at[pl.ds(base * CHUNK, CHUNK)],
            wsems.at[b]).wait()


_sc_gather = functools.partial(
    pl.kernel,
    mesh=plsc.VectorSubcoreMesh(core_axis_name="c", subcore_axis_name="s"),
    compiler_params=pltpu.CompilerParams(use_tc_tiling_on_sc=False),
    out_type=jax.ShapeDtypeStruct((EP, HID), jnp.float32),
    scratch_types=[
        pltpu.VMEM((G_CH, CHUNK), jnp.int32),
        pltpu.VMEM((NBUF, CHUNK, HID), jnp.float32),
        pltpu.VMEM_SHARED((N_NODES, HID), jnp.float32),
        pltpu.SemaphoreType.DMA((NBUF,)),
    ],
)(_gather_kernel)


def _scatter_kernel(m_hbm, dst_hbm, h_hbm, bias_hbm, out_hbm,
                    idx_v, buf, acc_v, h_v, bias_v, agg_sh, msems):
    sid = lax.axis_index("s")
    pltpu.sync_copy(dst_hbm.at[pl.ds(sid * S_CH, S_CH)], idx_v)

    z16 = jnp.zeros((16,), jnp.float32)

    def zbody(i, _):
        acc_v[i, pl.ds(0, 16)] = z16
        acc_v[i, pl.ds(16, 16)] = z16
        return 0

    lax.fori_loop(0, NPT, zbody, 0)
    pltpu.sync_copy(acc_v, agg_sh.at[pl.ds(sid * NPT, NPT)])
    plsc.subcore_barrier()

    for b in range(NBUF):
        pltpu.async_copy(m_hbm.at[pl.ds((sid * S_CH + b) * CHUNK, CHUNK)],
                         buf.at[b], msems.at[b])

    def sbody(g, _):
        for b in range(NBUF):
            j = g * NBUF + b
            src = m_hbm.at[pl.ds((sid * S_CH + j) * CHUNK, CHUNK)]
            pltpu.make_async_copy(src, buf.at[b], msems.at[b]).wait()
            pltpu.sync_copy(buf.at[b], agg_sh.at[idx_v.at[j]], add=True)

            @pl.when(j + NBUF < S_CH)
            def _():
                pltpu.async_copy(
                    m_hbm.at[pl.ds((sid * S_CH + j + NBUF) * CHUNK, CHUNK)],
                    buf.at[b], msems.at[b])
        return 0

    lax.fori_loop(0, S_CH // NBUF, sbody, 0)
    plsc.subcore_barrier()

    pltpu.sync_copy(agg_sh.at[pl.ds(sid * NT, NT)], acc_v.at[pl.ds(0, NT)])
    pltpu.sync_copy(h_hbm.at[pl.ds(sid * NT, NT)], h_v)
    pltpu.sync_copy(bias_hbm, bias_v)
    b0 = bias_v[pl.ds(0, 16)]
    b1 = bias_v[pl.ds(16, 16)]

    def ubody(i, _):
        acc_v[i, pl.ds(0, 16)] = acc_v[i, pl.ds(0, 16)] + h_v[i, pl.ds(0, 16)] + b0
        acc_v[i, pl.ds(16, 16)] = acc_v[i, pl.ds(16, 16)] + h_v[i, pl.ds(16, 16)] + b1
        return 0

    lax.fori_loop(0, NT, ubody, 0)
    pltpu.sync_copy(acc_v.at[pl.ds(0, NT)], out_hbm.at[pl.ds(sid * NT, NT)])


_sc_scatter = functools.partial(
    pl.kernel,
    mesh=plsc.VectorSubcoreMesh(core_axis_name="c", subcore_axis_name="s",
                                num_cores=1),
    compiler_params=pltpu.CompilerParams(use_tc_tiling_on_sc=False),
    out_type=jax.ShapeDtypeStruct((N_NODES, HID), jnp.float32),
    scratch_types=[
        pltpu.VMEM((S_CH, CHUNK), jnp.int32),
        pltpu.VMEM((NBUF, CHUNK, HID), jnp.float32),
        pltpu.VMEM((NPT, HID), jnp.float32),
        pltpu.VMEM((NT, HID), jnp.float32),
        pltpu.VMEM((HID,), jnp.float32),
        pltpu.VMEM_SHARED((NP, HID), jnp.float32),
        pltpu.SemaphoreType.DMA((NBUF,)),
    ],
)(_scatter_kernel)


def _mm_body(hs_ref, ef_ref, wcat_ref, r_ref, s_ref, bmat_ref, out_ref):
    hs8 = hs_ref[...]
    ef8 = ef_ref[...]
    u = jnp.dot(hs8, wcat_ref[...], preferred_element_type=jnp.float32)
    efrep = jnp.dot(ef8, r_ref[...], preferred_element_type=jnp.float32)
    out_ref[...] = (
        jnp.dot(u * efrep, s_ref[...], preferred_element_type=jnp.float32)
        + jnp.dot(hs8, bmat_ref[...], preferred_element_type=jnp.float32))


TE = 8192
TB = TE // 4


def _tc_matmul(hs4, ef4, wcat4, r4, s4, bmat4):
    return pl.pallas_call(
        _mm_body,
        grid=(EP // TE,),
        in_specs=[
            pl.BlockSpec((TB, 128), lambda i: (i, 0)),
            pl.BlockSpec((TB, 64), lambda i: (i, 0)),
            pl.BlockSpec((128, 2048), lambda i: (0, 0)),
            pl.BlockSpec((64, 2048), lambda i: (0, 0)),
            pl.BlockSpec((2048, 128), lambda i: (0, 0)),
            pl.BlockSpec((128, 128), lambda i: (0, 0)),
        ],
        out_specs=pl.BlockSpec((TB, 128), lambda i: (i, 0)),
        out_shape=jax.ShapeDtypeStruct((EP // 4, 128), jnp.float32),
    )(hs4, ef4, wcat4, r4, s4, bmat4)


def kernel(node_features, edge_index, initial_edge_features, W_ef, b_ef,
           bias0, bias1):
    E = initial_edge_features.shape[0]
    src = edge_index[0].astype(jnp.int32)
    dst = edge_index[1].astype(jnp.int32)
    src2d = jnp.pad(src, (0, EP - E)).reshape(EP // CHUNK, CHUNK)
    dst2d = jnp.pad(dst, (0, EP - E),
                    constant_values=N_NODES).reshape(EP // CHUNK, CHUNK)
    ef4 = jnp.pad(initial_edge_features.reshape(E // 4, 64),
                  ((0, (EP - E) // 4), (0, 0)))
    wcat = W_ef.reshape(D_EDGE, HID, HID).transpose(1, 0, 2).reshape(HID, 512)
    eye_d = jnp.eye(D_EDGE, dtype=jnp.float32)
    eye_h = jnp.eye(HID, dtype=jnp.float32)
    eye4 = jnp.eye(4, dtype=jnp.float32)
    rmat = jnp.repeat(eye_d, HID, axis=1).reshape(D_EDGE, 512)
    smat = jnp.tile(eye_h, (D_EDGE, 1))
    bmat = b_ef.reshape(HID, HID)
    wcat4 = jnp.kron(eye4, wcat)
    r4 = jnp.kron(eye4, rmat)
    s4 = jnp.kron(eye4, smat)
    bmat4 = jnp.kron(eye4, bmat)

    h = node_features
    for bias in (bias0, bias1):
        hs = _sc_gather(h, src2d)
        m4 = _tc_matmul(hs.reshape(EP // 4, 128), ef4, wcat4, r4, s4, bmat4)
        h = _sc_scatter(m4.reshape(EP, HID), dst2d, h, bias)
    return h
